# baseline (speedup 1.0000x reference)
import jax
import jax.numpy as jnp
from jax.experimental import pallas as pl

N_GRAPHS = 512


def _ident_kernel(x_ref, o_ref):
    o_ref[...] = x_ref[...]


def _gat(x, src, dst, ew, W, a_s, a_d, b):
    n = x.shape[0]
    h = x @ W
    e = (h @ a_s)[src] + (h @ a_d)[dst]
    e = jnp.where(e > 0, e, 0.2 * e)
    m = jax.ops.segment_max(e, dst, num_segments=n)
    m = jnp.where(jnp.isfinite(m), m, 0.0)
    ee = jnp.exp(e - m[dst])
    denom = jax.ops.segment_sum(ee, dst, num_segments=n)
    alpha = ee / jnp.maximum(denom[dst], 1e-16)
    msg = (alpha * ew)[:, None] * h[src]
    return jax.ops.segment_sum(msg, dst, num_segments=n) + b


def _graphnorm(x, w, b, ms, eps=1e-5):
    mean = jnp.mean(x, axis=0, keepdims=True)
    out = x - ms * mean
    var = jnp.mean(out * out, axis=0, keepdims=True)
    return out / jnp.sqrt(var + eps) * w + b


def kernel(x, edge_index, batch, edge_weight, params):
    p = params
    x = pl.pallas_call(
        _ident_kernel,
        out_shape=jax.ShapeDtypeStruct(x.shape, x.dtype),
        grid=(100,),
        in_specs=[pl.BlockSpec((1000, x.shape[1]), lambda i: (i, 0))],
        out_specs=pl.BlockSpec((1000, x.shape[1]), lambda i: (i, 0)),
    )(x)
    src, dst = edge_index[0], edge_index[1]
    x1 = jax.nn.relu(_graphnorm(_gat(x, src, dst, edge_weight, p["W1"], p["as1"], p["ad1"], p["b1"]), p["gnw1"], p["gnb1"], p["gnm1"]))
    xi = x1
    for l in (2, 3, 4):
        h = _graphnorm(_gat(xi, src, dst, edge_weight, p[f"W{l}"], p[f"as{l}"], p[f"ad{l}"], p[f"b{l}"]), p[f"gnw{l}"], p[f"gnb{l}"], p[f"gnm{l}"])
        xi = jax.nn.relu(h + xi)
    pooled = jax.ops.segment_sum(xi, batch, num_segments=N_GRAPHS)
    counts = jax.ops.segment_sum(jnp.ones((xi.shape[0],), jnp.float32), batch, num_segments=N_GRAPHS)
    pooled = pooled / jnp.maximum(counts, 1.0)[:, None]
    return pooled @ p["Wl"] + p["bl"]


# speedup vs baseline: 120.4717x; 120.4717x over previous
import functools

import jax
import jax.numpy as jnp
from jax import lax
from jax.experimental import pallas as pl
from jax.experimental.pallas import tpu as pltpu
from jax.experimental.pallas import tpu_sc as plsc

N = 100000
E = 6400000
H = 16
G = 512
NC, NS = 2, 16
NW = NC * NS
C = 512
K = C // 128
EPAD = ((E + NW * C - 1) // (NW * C)) * (NW * C)
ROWS = EPAD // 128
RW = ROWS // NW
NCHUNK = RW // K
BN = 1024
NPAD = 98 * BN
NB = NPAD // BN
RPT = NPAD // NS

def _sc_edge_body(src_hbm, dst_hbm, ew_hbm, h_hbm, hs_hbm, hd_hbm, zacc, zden,
                  acc_out, den_out,
                  acc_sh, den_sh,
                  src_c, dst_c, ew_c, hsg, hdg, wbuf, rows,
                  gsem, tsem, ssem):
    cid = lax.axis_index("c")
    sid = lax.axis_index("s")
    nb = sid * RPT
    pltpu.sync_copy(zacc.at[pl.ds(nb, RPT)], acc_sh.at[pl.ds(nb, RPT)])
    pltpu.sync_copy(zden.at[pl.ds(nb, RPT)], den_sh.at[pl.ds(nb, RPT)])
    plsc.subcore_barrier()

    wbase = (cid * NS + sid) * RW

    def chunk(t, carry):
        rb = wbase + t * K
        pltpu.sync_copy(src_hbm.at[pl.ds(rb, K)], src_c)
        pltpu.sync_copy(dst_hbm.at[pl.ds(rb, K)], dst_c)
        pltpu.sync_copy(ew_hbm.at[pl.ds(rb, K)], ew_c)
        cps = []
        for j in range(K):
            cps.append(pltpu.async_copy(hs_hbm.at[src_c.at[j]], hsg.at[j], tsem))
            cps.append(pltpu.async_copy(hd_hbm.at[dst_c.at[j]], hdg.at[j], tsem))
            cps.append(pltpu.async_copy(
                h_hbm.at[src_c.at[j]], rows.at[pl.ds(j * 128, 128)], gsem))
        for cp in cps:
            cp.wait()

        def vstep(r, c2):
            j = r // 8
            col = (r % 8) * 16
            e = hsg[j, pl.ds(col, 16)] + hdg[j, pl.ds(col, 16)]
            e = jnp.where(e > 0.0, e, 0.2 * e)
            ee = jnp.exp(e)
            wbuf[j, pl.ds(col, 16)] = ee
            w = ee * ew_c[j, pl.ds(col, 16)]
            base = r * 16
            for l in range(16):
                rows[base + l, :] = rows[base + l, :] * w[l]
            return c2

        lax.fori_loop(0, C // 16, vstep, 0)

        sps = []
        for j in range(K):
            sps.append(pltpu.async_copy(
                rows.at[pl.ds(j * 128, 128)], acc_sh.at[dst_c.at[j]], ssem,
                add=True))
            sps.append(pltpu.async_copy(
                wbuf.at[j], den_sh.at[dst_c.at[j]], ssem, add=True))
        for sp in sps:
            sp.wait()
        return carry

    lax.fori_loop(0, NCHUNK, chunk, 0)

    plsc.subcore_barrier()
    pltpu.sync_copy(acc_sh.at[pl.ds(nb, RPT)], acc_out.at[cid, pl.ds(nb, RPT)])
    pltpu.sync_copy(den_sh.at[pl.ds(nb, RPT)], den_out.at[cid, pl.ds(nb, RPT)])


@functools.lru_cache(maxsize=1)
def _make_sc_edge():
    mesh = plsc.VectorSubcoreMesh(
        core_axis_name="c", subcore_axis_name="s",
        num_cores=NC, num_subcores=NS)
    return pl.kernel(
        _sc_edge_body,
        out_type=(
            jax.ShapeDtypeStruct((NC, NPAD, H), jnp.float32),
            jax.ShapeDtypeStruct((NC, NPAD), jnp.float32),
        ),
        mesh=mesh,
        compiler_params=pltpu.CompilerParams(
            use_tc_tiling_on_sc=False, needs_layout_passes=False),
        scratch_types=[
        pltpu.VMEM_SHARED((NPAD, H), jnp.float32),
        pltpu.VMEM_SHARED((NPAD,), jnp.float32),
        pltpu.VMEM((K, 128), jnp.int32),
        pltpu.VMEM((K, 128), jnp.int32),
        pltpu.VMEM((K, 128), jnp.float32),
        pltpu.VMEM((K, 128), jnp.float32),
        pltpu.VMEM((K, 128), jnp.float32),
        pltpu.VMEM((K, 128), jnp.float32),
        pltpu.VMEM((C, H), jnp.float32),
            pltpu.SemaphoreType.DMA,
            pltpu.SemaphoreType.DMA,
            pltpu.SemaphoreType.DMA,
        ],
    )


def _pre_body(x_ref, w_ref, as_ref, ad_ref, h_ref, hs_ref, hd_ref):
    h = jnp.dot(x_ref[...], w_ref[...], preferred_element_type=jnp.float32)
    h_ref[...] = h
    hs_ref[...] = jnp.dot(h, as_ref[...], preferred_element_type=jnp.float32)
    hd_ref[...] = jnp.dot(h, ad_ref[...], preferred_element_type=jnp.float32)


def _make_pre(fin):
    return pl.pallas_call(
        _pre_body,
        grid=(NB,),
        in_specs=[
            pl.BlockSpec((BN, fin), lambda i: (i, 0)),
            pl.BlockSpec((fin, H), lambda i: (0, 0)),
            pl.BlockSpec((H, 1), lambda i: (0, 0)),
            pl.BlockSpec((H, 1), lambda i: (0, 0)),
        ],
        out_specs=[
            pl.BlockSpec((BN, H), lambda i: (i, 0)),
            pl.BlockSpec((BN, 1), lambda i: (i, 0)),
            pl.BlockSpec((BN, 1), lambda i: (i, 0)),
        ],
        out_shape=[
            jax.ShapeDtypeStruct((NPAD, H), jnp.float32),
            jax.ShapeDtypeStruct((NPAD, 1), jnp.float32),
            jax.ShapeDtypeStruct((NPAD, 1), jnp.float32),
        ],
    )


def _combine_body(a0_ref, a1_ref, dt_ref, b_ref, y_ref, s_ref):
    i = pl.program_id(0)
    z = a0_ref[...] + a1_ref[...]
    d = dt_ref[:, 0:1] + dt_ref[:, 1:2]
    y = z / jnp.maximum(d, 1e-16) + b_ref[...]
    y_ref[...] = y
    rowi = i * BN + lax.broadcasted_iota(jnp.int32, (BN, 1), 0)
    ym = jnp.where(rowi < N, y, 0.0)
    s = jnp.concatenate(
        [jnp.sum(ym, axis=0, keepdims=True),
         jnp.sum(ym * ym, axis=0, keepdims=True)], axis=0)

    @pl.when(i == 0)
    def _():
        s_ref[...] = s

    @pl.when(i > 0)
    def _():
        s_ref[...] = s_ref[...] + s


_combine = pl.pallas_call(
    _combine_body,
    grid=(NB,),
    in_specs=[
        pl.BlockSpec((BN, H), lambda i: (i, 0)),
        pl.BlockSpec((BN, H), lambda i: (i, 0)),
        pl.BlockSpec((BN, NC), lambda i: (i, 0)),
        pl.BlockSpec((1, H), lambda i: (0, 0)),
    ],
    out_specs=[
        pl.BlockSpec((BN, H), lambda i: (i, 0)),
        pl.BlockSpec((2, H), lambda i: (0, 0)),
    ],
    out_shape=[
        jax.ShapeDtypeStruct((NPAD, H), jnp.float32),
        jax.ShapeDtypeStruct((2, H), jnp.float32),
    ],
)


def _norm_body(residual, nxt, *refs):
    if residual:
        (y_ref, s_ref, w_ref, b_ref, ms_ref, xp_ref), refs = refs[:6], refs[6:]
    else:
        (y_ref, s_ref, w_ref, b_ref, ms_ref), refs = refs[:5], refs[5:]
        xp_ref = None
    if nxt:
        (wn_ref, asn_ref, adn_ref), refs = refs[:3], refs[3:]
        xi_ref, h_ref, hs_ref, hd_ref = refs
    else:
        (xi_ref,) = refs
    i = pl.program_id(0)
    inv_n = 1.0 / N
    mean = s_ref[0:1, :] * inv_n
    ey2 = s_ref[1:2, :] * inv_n
    ms = ms_ref[...]
    var = ey2 - ms * (2.0 - ms) * mean * mean
    out = (y_ref[...] - ms * mean) * lax.rsqrt(var + 1e-5) * w_ref[...] + b_ref[...]
    if xp_ref is not None:
        out = out + xp_ref[...]
    z = jnp.maximum(out, 0.0)
    rowi = i * BN + lax.broadcasted_iota(jnp.int32, (BN, 1), 0)
    xi = jnp.where(rowi < N, z, 0.0)
    xi_ref[...] = xi
    if nxt:
        h = jnp.dot(xi, wn_ref[...], preferred_element_type=jnp.float32)
        h_ref[...] = h
        hs_ref[...] = jnp.dot(h, asn_ref[...], preferred_element_type=jnp.float32)
        hd_ref[...] = jnp.dot(h, adn_ref[...], preferred_element_type=jnp.float32)


def _make_norm(residual, nxt):
    in_specs = [
        pl.BlockSpec((BN, H), lambda i: (i, 0)),
        pl.BlockSpec((2, H), lambda i: (0, 0)),
        pl.BlockSpec((1, H), lambda i: (0, 0)),
        pl.BlockSpec((1, H), lambda i: (0, 0)),
        pl.BlockSpec((1, H), lambda i: (0, 0)),
    ]
    if residual:
        in_specs.append(pl.BlockSpec((BN, H), lambda i: (i, 0)))
    out_specs = [pl.BlockSpec((BN, H), lambda i: (i, 0))]
    out_shape = [jax.ShapeDtypeStruct((NPAD, H), jnp.float32)]
    if nxt:
        in_specs += [
            pl.BlockSpec((H, H), lambda i: (0, 0)),
            pl.BlockSpec((H, 1), lambda i: (0, 0)),
            pl.BlockSpec((H, 1), lambda i: (0, 0)),
        ]
        out_specs += [
            pl.BlockSpec((BN, H), lambda i: (i, 0)),
            pl.BlockSpec((BN, 1), lambda i: (i, 0)),
            pl.BlockSpec((BN, 1), lambda i: (i, 0)),
        ]
        out_shape += [
            jax.ShapeDtypeStruct((NPAD, H), jnp.float32),
            jax.ShapeDtypeStruct((NPAD, 1), jnp.float32),
            jax.ShapeDtypeStruct((NPAD, 1), jnp.float32),
        ]
    return pl.pallas_call(
        functools.partial(_norm_body, residual, nxt),
        grid=(NB,),
        in_specs=in_specs,
        out_specs=out_specs,
        out_shape=out_shape,
    )


def _pool_body(xi_ref, b_ref, wl_ref, bl_ref, pooled_ref, cnt_ref, out_ref):
    i = pl.program_id(0)
    xb = xi_ref[...].reshape(BN, H)
    bm = b_ref[...].reshape(1, BN)
    gi = lax.broadcasted_iota(jnp.int32, (G, BN), 0)
    oh = (gi == bm).astype(jnp.float32)
    pool_blk = jnp.dot(oh, xb, preferred_element_type=jnp.float32)
    cnt_blk = jnp.sum(oh, axis=1, keepdims=True)

    @pl.when(i == 0)
    def _():
        pooled_ref[...] = pool_blk
        cnt_ref[...] = cnt_blk

    @pl.when(i > 0)
    def _():
        pooled_ref[...] = pooled_ref[...] + pool_blk
        cnt_ref[...] = cnt_ref[...] + cnt_blk

    @pl.when(i == NB - 1)
    def _():
        pooled = pooled_ref[...] / jnp.maximum(cnt_ref[...], 1.0)
        out_ref[...] = jnp.dot(
            pooled, wl_ref[...], preferred_element_type=jnp.float32) + bl_ref[...]


_pool = pl.pallas_call(
    _pool_body,
    grid=(NB,),
    in_specs=[
        pl.BlockSpec((1, BN, H), lambda i: (i, 0, 0)),
        pl.BlockSpec((1, 1, BN), lambda i: (i, 0, 0)),
        pl.BlockSpec((H, 2), lambda i: (0, 0)),
        pl.BlockSpec((1, 2), lambda i: (0, 0)),
    ],
    out_specs=[
        pl.BlockSpec((G, H), lambda i: (0, 0)),
        pl.BlockSpec((G, 1), lambda i: (0, 0)),
        pl.BlockSpec((G, 2), lambda i: (0, 0)),
    ],
    out_shape=[
        jax.ShapeDtypeStruct((G, H), jnp.float32),
        jax.ShapeDtypeStruct((G, 1), jnp.float32),
        jax.ShapeDtypeStruct((G, 2), jnp.float32),
    ],
)


def kernel(x, edge_index, batch, edge_weight, params):
    p = params
    npe = NPAD - N
    pade = EPAD - E
    pad_idx = (N + (jnp.arange(pade, dtype=jnp.int32) % 128)).astype(jnp.int32)
    src2 = jnp.concatenate([edge_index[0], pad_idx]).reshape(ROWS, 128)
    dst2 = jnp.concatenate([edge_index[1], pad_idx]).reshape(ROWS, 128)
    ew2 = jnp.concatenate(
        [edge_weight, jnp.zeros((pade,), jnp.float32)]).reshape(ROWS, 128)
    zacc = jnp.zeros((NPAD, H), jnp.float32)
    zden = jnp.zeros((NPAD,), jnp.float32)

    xpad = jnp.pad(x, ((0, npe), (0, 0)))
    h, hs, hd = _make_pre(x.shape[1])(
        xpad, p["W1"], p["as1"].reshape(H, 1), p["ad1"].reshape(H, 1))

    xi = None
    for l in (1, 2, 3, 4):
        acc, den = _make_sc_edge()(
            src2, dst2, ew2, h, hs.reshape(NPAD), hd.reshape(NPAD), zacc, zden)
        y, sums = _combine(
            acc[0], acc[1], den.T, p[f"b{l}"].reshape(1, H))
        gw = p[f"gnw{l}"].reshape(1, H)
        gb = p[f"gnb{l}"].reshape(1, H)
        gm = p[f"gnm{l}"].reshape(1, H)
        if l < 4:
            wn = p[f"W{l + 1}"]
            asn = p[f"as{l + 1}"].reshape(H, 1)
            adn = p[f"ad{l + 1}"].reshape(H, 1)
            if l == 1:
                xi, h, hs, hd = _make_norm(False, True)(
                    y, sums, gw, gb, gm, wn, asn, adn)
            else:
                xi, h, hs, hd = _make_norm(True, True)(
                    y, sums, gw, gb, gm, xi, wn, asn, adn)
        else:
            (xi,) = _make_norm(True, False)(y, sums, gw, gb, gm, xi)

    batch_p = jnp.pad(batch, (0, npe), constant_values=1000)
    pooled, cnt, logits = _pool(
        xi.reshape(NB, BN, H), batch_p.reshape(NB, 1, BN),
        p["Wl"], p["bl"].reshape(1, 2))
    return logits
